# baseline (speedup 1.0000x reference)
import functools

import jax
import jax.numpy as jnp
from jax import lax
from jax.experimental import pallas as pl
from jax.experimental.pallas import tpu as pltpu

EPS = 1e-5


def _round_up(x, m):
    return (x + m - 1) // m * m


def _fused_kernel(feat_ref, w_ref, w2_ref, p_ref, out_ref,
                  fbf_ref, c_ref, rs_ref, st_ref, *, n1, tm, n_rows):
    i = pl.program_id(0)
    units = w2_ref.shape[0]
    f_sz = feat_ref.shape[1]

    @pl.when(i < n1)
    def _gram_phase():
        @pl.when(i == 0)
        def _():
            c_ref[...] = jnp.zeros_like(c_ref)
            rs_ref[...] = jnp.zeros_like(rs_ref)

        fb = feat_ref[...].astype(jnp.bfloat16)
        fbf_ref[pl.ds(i * tm, tm), :] = fb
        c_ref[...] += lax.dot_general(
            fb, fb, (((0,), (0,)), ((), ())),
            preferred_element_type=jnp.float32)
        rs_ref[...] += jnp.sum(
            fb.astype(jnp.float32).reshape(tm // 8, 8, f_sz), axis=0)

    @pl.when(i == n1)
    def _stats():
        inv_n = 1.0 / n_rows
        w1t = w_ref[:, :units].astype(jnp.float32)
        d = jnp.dot(c_ref[...], w1t, preferred_element_type=jnp.float32)
        e2 = jnp.sum(w1t * d, axis=0, keepdims=True) * inv_n
        m8 = jnp.dot(rs_ref[...], w1t, preferred_element_type=jnp.float32)
        mean = jnp.sum(m8, axis=0, keepdims=True) * inv_n
        var = e2 - mean * mean
        inv_std = lax.rsqrt(jnp.maximum(var, 0.0) + EPS)
        a = inv_std * p_ref[1:2, :]
        st_ref[0:1, :] = a
        st_ref[1:2, :] = p_ref[2:3, :] - mean * a

    @pl.when(i >= n1)
    def _apply_phase():
        j = i - n1
        fb = fbf_ref[pl.ds(j * tm, tm), :]
        hs = jnp.dot(fb, w_ref[...], preferred_element_type=jnp.float32)
        h = hs[:, :units]
        s = hs[:, units:]

        h = h * st_ref[0:1, :] + st_ref[1:2, :]
        hb = jnp.maximum(h, 0.0).astype(jnp.bfloat16)

        f = (jnp.dot(hb, w2_ref[...], preferred_element_type=jnp.float32)
             + s + p_ref[0:1, :])

        mu = jnp.mean(f, axis=-1, keepdims=True)
        d = f - mu
        v = jnp.mean(d * d, axis=-1, keepdims=True)
        out_ref[...] = (d * lax.rsqrt(v + EPS) * p_ref[3:4, :]
                        + p_ref[4:5, :]).astype(out_ref.dtype)


def kernel(feat, w1, w2, b2, ws, bs, bn_gamma, bn_beta, ln_gamma, ln_beta):
    n, f_sz = feat.shape
    u = w2.shape[0]

    w_feat = jnp.concatenate([w1.T, ws.T], axis=1).astype(jnp.bfloat16)
    w2t = w2.T.astype(jnp.bfloat16)
    pvec = jnp.zeros((8, u), jnp.float32)
    pvec = pvec.at[0].set(b2 + bs)
    pvec = pvec.at[1].set(bn_gamma)
    pvec = pvec.at[2].set(bn_beta)
    pvec = pvec.at[3].set(ln_gamma)
    pvec = pvec.at[4].set(ln_beta)

    tm = 1024
    n_pad = _round_up(n, tm)
    feat_p = jnp.pad(feat, ((0, n_pad - n), (0, 0))) if n_pad != n else feat
    n1 = n_pad // tm

    out = pl.pallas_call(
        functools.partial(_fused_kernel, n1=n1, tm=tm, n_rows=float(n)),
        out_shape=jax.ShapeDtypeStruct((n_pad, u), feat.dtype),
        grid=(2 * n1,),
        in_specs=[
            pl.BlockSpec((tm, f_sz), lambda i: (jnp.minimum(i, n1 - 1), 0)),
            pl.BlockSpec((f_sz, 2 * u), lambda i: (0, 0)),
            pl.BlockSpec((u, u), lambda i: (0, 0)),
            pl.BlockSpec((8, u), lambda i: (0, 0)),
        ],
        out_specs=pl.BlockSpec(
            (tm, u), lambda i: (jnp.maximum(i - n1, 0), 0)),
        scratch_shapes=[
            pltpu.VMEM((n_pad, f_sz), jnp.bfloat16),
            pltpu.VMEM((f_sz, f_sz), jnp.float32),
            pltpu.VMEM((8, f_sz), jnp.float32),
            pltpu.VMEM((8, u), jnp.float32),
        ],
        compiler_params=pltpu.CompilerParams(
            dimension_semantics=("arbitrary",),
            vmem_limit_bytes=48 * 1024 * 1024),
    )(feat_p, w_feat, w2t, pvec)

    return out[:n] if n_pad != n else out
